# baseline (speedup 1.0000x reference)
import functools

import jax
import jax.numpy as jnp
from jax import lax
from jax.experimental import pallas as pl
from jax.experimental.pallas import tpu as pltpu

B = 16
P = 32768
G = 32
R = 256
L = 128
CH = 32
NC = R // CH
THR = 0.35
NEGPOS = 7
VAR0 = 0.1
VAR1 = 0.2
INF_BITS = 0x7F800000


def _sl1(d):
    a = jnp.abs(d)
    return jnp.where(a < 1.0, 0.5 * a * a, a - 0.5)


def _body(gtb_ref, gtl_ref, gtn_ref, img_ref, locT_ref, confT_ref, landmT_ref,
          priT_ref, out_ref, vb_ref, ks_ref):
    b = pl.program_id(0)

    @pl.when(b == 0)
    def _init():
        out_ref[0] = 0.0
        out_ref[1] = 0.0
        out_ref[2] = 0.0
        out_ref[3] = 0.0

    w = img_ref[0, 0, 1].astype(jnp.float32)
    h = img_ref[0, 0, 0].astype(jnp.float32)
    g = gtn_ref[0, 0, 0]

    tx1 = [gtb_ref[0, j, 0] / w for j in range(G)]
    ty1 = [gtb_ref[0, j, 1] / h for j in range(G)]
    tx2 = [gtb_ref[0, j, 2] / w for j in range(G)]
    ty2 = [gtb_ref[0, j, 3] / h for j in range(G)]
    ta = [(tx2[j] - tx1[j]) * (ty2[j] - ty1[j]) for j in range(G)]

    bpoM = [jnp.float32(-1.0)] * G
    bpiM = [jnp.int32(P)] * G
    best_l = []
    bti_l = []
    for c in range(NC):
        r0 = c * CH
        pcx = priT_ref[0, r0:r0 + CH, :]
        pcy = priT_ref[1, r0:r0 + CH, :]
        pw = priT_ref[2, r0:r0 + CH, :]
        ph = priT_ref[3, r0:r0 + CH, :]
        px1 = pcx - pw * 0.5
        py1 = pcy - ph * 0.5
        px2 = pcx + pw * 0.5
        py2 = pcy + ph * 0.5
        parea = pw * ph
        lin = (lax.broadcasted_iota(jnp.int32, (CH, L), 0) * L
               + lax.broadcasted_iota(jnp.int32, (CH, L), 1) + r0 * L)
        best = jnp.full((CH, L), -2.0, jnp.float32)
        bti = jnp.zeros((CH, L), jnp.int32)
        for j in range(G):
            iw = jnp.maximum(jnp.minimum(tx2[j], px2) - jnp.maximum(tx1[j], px1), 0.0)
            ih = jnp.maximum(jnp.minimum(ty2[j], py2) - jnp.maximum(ty1[j], py1), 0.0)
            inter = iw * ih
            iou = inter / (ta[j] + parea - inter)
            cm = jnp.max(iou)
            ci = jnp.min(jnp.where(iou == cm, lin, P))
            gtr = cm > bpoM[j]
            bpoM[j] = jnp.where(gtr, cm, bpoM[j])
            bpiM[j] = jnp.where(gtr, ci, bpiM[j])
            iou_eff = jnp.where(j < g, iou, -2.0)
            upd = iou_eff > best
            best = jnp.where(upd, iou_eff, best)
            bti = jnp.where(upd, j, bti)
        best_l.append(best)
        bti_l.append(bti)

    t1 = [jnp.where((j < g) & (bpoM[j] >= 0.2), bpiM[j], -1) for j in range(G)]
    t2 = [jnp.where(j < g, bpiM[j], -2) for j in range(G)]

    ll_img = jnp.float32(0.0)
    llm_img = jnp.float32(0.0)
    lc_pos = jnp.float32(0.0)
    npos_i = jnp.int32(0)
    for c in range(NC):
        r0 = c * CH
        best = best_l[c]
        bti = bti_l[c]
        lin = (lax.broadcasted_iota(jnp.int32, (CH, L), 0) * L
               + lax.broadcasted_iota(jnp.int32, (CH, L), 1) + r0 * L)
        forced = jnp.zeros((CH, L), jnp.bool_)
        jl = jnp.full((CH, L), -1, jnp.int32)
        for j in range(G):
            forced = forced | (lin == t1[j])
            jl = jnp.where(lin == t2[j], j, jl)
        bti = jnp.where(jl >= 0, jl, bti)
        pos = (best >= THR) | forced

        mcx = jnp.zeros((CH, L), jnp.float32)
        mcy = jnp.zeros((CH, L), jnp.float32)
        mw = jnp.ones((CH, L), jnp.float32)
        mh = jnp.ones((CH, L), jnp.float32)
        mlm = [jnp.zeros((CH, L), jnp.float32) for _ in range(10)]
        for j in range(G):
            sel = bti == j
            mcx = jnp.where(sel, (tx1[j] + tx2[j]) * 0.5, mcx)
            mcy = jnp.where(sel, (ty1[j] + ty2[j]) * 0.5, mcy)
            mw = jnp.where(sel, tx2[j] - tx1[j], mw)
            mh = jnp.where(sel, ty2[j] - ty1[j], mh)
            for k in range(5):
                mlm[2 * k] = jnp.where(sel, gtl_ref[0, j, 2 * k] / w, mlm[2 * k])
                mlm[2 * k + 1] = jnp.where(sel, gtl_ref[0, j, 2 * k + 1] / h,
                                           mlm[2 * k + 1])

        pcx = priT_ref[0, r0:r0 + CH, :]
        pcy = priT_ref[1, r0:r0 + CH, :]
        pw = priT_ref[2, r0:r0 + CH, :]
        ph = priT_ref[3, r0:r0 + CH, :]
        inv_vw = 1.0 / (VAR0 * pw)
        inv_vh = 1.0 / (VAR0 * ph)
        acc = _sl1(locT_ref[0, 0, r0:r0 + CH, :] - (mcx - pcx) * inv_vw)
        acc = acc + _sl1(locT_ref[0, 1, r0:r0 + CH, :] - (mcy - pcy) * inv_vh)
        acc = acc + _sl1(locT_ref[0, 2, r0:r0 + CH, :] - jnp.log(mw / pw) * (1.0 / VAR1))
        acc = acc + _sl1(locT_ref[0, 3, r0:r0 + CH, :] - jnp.log(mh / ph) * (1.0 / VAR1))
        ll_img = ll_img + jnp.sum(jnp.where(pos, acc, 0.0))

        lacc = jnp.zeros((CH, L), jnp.float32)
        for k in range(5):
            lacc = lacc + _sl1(landmT_ref[0, 2 * k, r0:r0 + CH, :]
                               - (mlm[2 * k] - pcx) * inv_vw)
            lacc = lacc + _sl1(landmT_ref[0, 2 * k + 1, r0:r0 + CH, :]
                               - (mlm[2 * k + 1] - pcy) * inv_vh)
        llm_img = llm_img + jnp.sum(jnp.where(pos, lacc, 0.0))

        c0 = confT_ref[0, 0, r0:r0 + CH, :]
        c1 = confT_ref[0, 1, r0:r0 + CH, :]
        mx = jnp.maximum(c0, c1)
        lse = mx + jnp.log(jnp.exp(c0 - mx) + jnp.exp(c1 - mx))
        gath = jnp.where(pos, c1, c0)
        v = lse - gath
        lc_pos = lc_pos + jnp.sum(jnp.where(pos, v, 0.0))
        npos_i = npos_i + jnp.sum(pos.astype(jnp.int32))

        vb_ref[b, r0:r0 + CH, :] = jnp.where(
            pos, -1, lax.bitcast_convert_type(v, jnp.int32))

    npos = npos_i.astype(jnp.float32)
    k_sel = jnp.minimum(NEGPOS * npos_i, P - npos_i)
    ks_ref[b] = k_sel

    out_ref[0] += ll_img
    out_ref[1] += lc_pos
    out_ref[2] += llm_img
    out_ref[3] += npos

    @pl.when(b == B - 1)
    def _final():
        ks = [ks_ref[i] for i in range(B)]
        k1 = [jnp.maximum(ks[i], 1) for i in range(B)]

        def bs_body(_, carry):
            lo, hi = carry
            nlo = []
            nhi = []
            for i in range(B):
                mid = lo[i] + (hi[i] - lo[i]) // 2
                cnt = jnp.sum((vb_ref[i] > mid).astype(jnp.int32))
                gek = cnt >= k1[i]
                nlo.append(jnp.where(gek, mid, lo[i]))
                nhi.append(jnp.where(gek, hi[i], mid))
            return tuple(nlo), tuple(nhi)

        lo0 = tuple(jnp.int32(-1) for _ in range(B))
        hi0 = tuple(jnp.int32(INF_BITS) for _ in range(B))
        _, hi = lax.fori_loop(0, 31, bs_body, (lo0, hi0))

        lc_neg = jnp.float32(0.0)
        for i in range(B):
            vb = vb_ref[i]
            t = lax.bitcast_convert_type(hi[i], jnp.float32)
            gt_mask = vb > hi[i]
            cnt_gt = jnp.sum(gt_mask.astype(jnp.int32))
            vi = lax.bitcast_convert_type(vb, jnp.float32)
            sum_gt = jnp.sum(jnp.where(gt_mask, vi, 0.0))
            topsum = sum_gt + (k1[i] - cnt_gt).astype(jnp.float32) * t
            lc_neg = lc_neg + jnp.where(ks[i] > 0, topsum, 0.0)

        n = jnp.maximum(out_ref[3], 1.0)
        out_ref[0] = out_ref[0] / n
        out_ref[1] = (out_ref[1] + lc_neg) / n
        out_ref[2] = out_ref[2] / n


@jax.jit
def _run(loc_data, conf_data, landm_data, priors, gt_bboxes, gt_landmarks,
         gt_num, img_shape):
    locT = loc_data.transpose(0, 2, 1).reshape(B, 4, R, L)
    confT = conf_data.transpose(0, 2, 1).reshape(B, 2, R, L)
    landmT = landm_data.transpose(0, 2, 1).reshape(B, 10, R, L)
    priT = priors.T.reshape(4, R, L)

    out = pl.pallas_call(
        _body,
        grid=(B,),
        in_specs=[
            pl.BlockSpec((1, G, 4), lambda b: (b, 0, 0), memory_space=pltpu.SMEM),
            pl.BlockSpec((1, G, 10), lambda b: (b, 0, 0), memory_space=pltpu.SMEM),
            pl.BlockSpec((1, 1, 1), lambda b: (b, 0, 0), memory_space=pltpu.SMEM),
            pl.BlockSpec((1, 1, 2), lambda b: (b, 0, 0), memory_space=pltpu.SMEM),
            pl.BlockSpec((1, 4, R, L), lambda b: (b, 0, 0, 0)),
            pl.BlockSpec((1, 2, R, L), lambda b: (b, 0, 0, 0)),
            pl.BlockSpec((1, 10, R, L), lambda b: (b, 0, 0, 0)),
            pl.BlockSpec((4, R, L), lambda b: (0, 0, 0)),
        ],
        out_specs=pl.BlockSpec(memory_space=pltpu.SMEM),
        out_shape=jax.ShapeDtypeStruct((4,), jnp.float32),
        scratch_shapes=[pltpu.VMEM((B, R, L), jnp.int32),
                        pltpu.SMEM((B,), jnp.int32)],
    )(gt_bboxes, gt_landmarks, gt_num.reshape(B, 1, 1),
      img_shape.reshape(B, 1, 2), locT, confT, landmT, priT)
    return out[0], out[1], out[2]


def kernel(loc_data, conf_data, landm_data, priors, gt_bboxes, gt_labels,
           gt_landmarks, gt_num, img_shape):
    del gt_labels
    return _run(loc_data, conf_data, landm_data, priors, gt_bboxes.astype(jnp.float32),
                gt_landmarks.astype(jnp.float32), gt_num, img_shape)


# speedup vs baseline: 3.5780x; 3.5780x over previous
import functools

import jax
import jax.numpy as jnp
from jax import lax
from jax.experimental import pallas as pl
from jax.experimental.pallas import tpu as pltpu

B = 16
P = 32768
G = 32
R = 256
L = 128
CH = 32
NC = R // CH
THR = 0.35
NEGPOS = 7
VAR0 = 0.1
VAR1 = 0.2
INF_BITS = 0x7F800000


def _sl1(d):
    a = jnp.abs(d)
    return jnp.where(a < 1.0, 0.5 * a * a, a - 0.5)


def _body(gtb_ref, gtl_ref, gtn_ref, img_ref, locT_ref, confT_ref, landmT_ref,
          priT_ref, out_ref, vb_ref, ks_ref, best_ref, bti_ref, forced_ref,
          jl_ref):
    b = pl.program_id(0)

    @pl.when(b == 0)
    def _init():
        out_ref[0] = 0.0
        out_ref[1] = 0.0
        out_ref[2] = 0.0
        out_ref[3] = 0.0

    w = img_ref[0, 0, 1].astype(jnp.float32)
    h = img_ref[0, 0, 0].astype(jnp.float32)
    g = gtn_ref[0, 0, 0]

    pcx = priT_ref[0]
    pcy = priT_ref[1]
    pw = priT_ref[2]
    ph = priT_ref[3]
    px1 = pcx - pw * 0.5
    py1 = pcy - ph * 0.5
    px2 = pcx + pw * 0.5
    py2 = pcy + ph * 0.5
    parea = pw * ph

    lin = (lax.broadcasted_iota(jnp.int32, (R, L), 0) * L
           + lax.broadcasted_iota(jnp.int32, (R, L), 1))

    tx1 = [gtb_ref[0, j, 0] / w for j in range(G)]
    ty1 = [gtb_ref[0, j, 1] / h for j in range(G)]
    tx2 = [gtb_ref[0, j, 2] / w for j in range(G)]
    ty2 = [gtb_ref[0, j, 3] / h for j in range(G)]

    best_ref[...] = jnp.full((R, L), -2.0, jnp.float32)
    bti_ref[...] = jnp.zeros((R, L), jnp.int32)
    forced_ref[...] = jnp.zeros((R, L), jnp.int32)
    jl_ref[...] = jnp.full((R, L), -1, jnp.int32)
    for j in range(G):
        @pl.when(j < g)
        def _match(j=j):
            iw = jnp.maximum(jnp.minimum(tx2[j], px2) - jnp.maximum(tx1[j], px1), 0.0)
            ih = jnp.maximum(jnp.minimum(ty2[j], py2) - jnp.maximum(ty1[j], py1), 0.0)
            inter = iw * ih
            ta = (tx2[j] - tx1[j]) * (ty2[j] - ty1[j])
            iou = inter / (ta + parea - inter)
            bpo_j = jnp.max(iou)
            bpi_j = jnp.min(jnp.where(iou == bpo_j, lin, P))
            upd = iou > best_ref[...]
            best_ref[...] = jnp.where(upd, iou, best_ref[...])
            bti_ref[...] = jnp.where(upd, j, bti_ref[...])
            t1 = jnp.where(bpo_j >= 0.2, bpi_j, -1)
            forced_ref[...] = jnp.where(lin == t1, 1, forced_ref[...])
            jl_ref[...] = jnp.where(lin == bpi_j, j, jl_ref[...])

    best = best_ref[...]
    jl = jl_ref[...]
    bti = jnp.where(jl >= 0, jl, bti_ref[...])
    forced = forced_ref[...] != 0

    pos = (best >= THR) | forced
    npos = jnp.sum(pos.astype(jnp.float32))

    mcx = jnp.zeros((R, L), jnp.float32)
    mcy = jnp.zeros((R, L), jnp.float32)
    mw = jnp.ones((R, L), jnp.float32)
    mh = jnp.ones((R, L), jnp.float32)
    mlm = [jnp.zeros((R, L), jnp.float32) for _ in range(10)]
    for j in range(G):
        sel = bti == j
        mcx = jnp.where(sel, (tx1[j] + tx2[j]) * 0.5, mcx)
        mcy = jnp.where(sel, (ty1[j] + ty2[j]) * 0.5, mcy)
        mw = jnp.where(sel, tx2[j] - tx1[j], mw)
        mh = jnp.where(sel, ty2[j] - ty1[j], mh)
        for k in range(5):
            mlm[2 * k] = jnp.where(sel, gtl_ref[0, j, 2 * k] / w, mlm[2 * k])
            mlm[2 * k + 1] = jnp.where(sel, gtl_ref[0, j, 2 * k + 1] / h, mlm[2 * k + 1])

    inv_vw = 1.0 / (VAR0 * pw)
    inv_vh = 1.0 / (VAR0 * ph)
    acc = _sl1(locT_ref[0, 0] - (mcx - pcx) * inv_vw)
    acc = acc + _sl1(locT_ref[0, 1] - (mcy - pcy) * inv_vh)
    acc = acc + _sl1(locT_ref[0, 2] - jnp.log(mw / pw) * (1.0 / VAR1))
    acc = acc + _sl1(locT_ref[0, 3] - jnp.log(mh / ph) * (1.0 / VAR1))
    ll_img = jnp.sum(jnp.where(pos, acc, 0.0))

    lacc = jnp.zeros((R, L), jnp.float32)
    for k in range(5):
        lacc = lacc + _sl1(landmT_ref[0, 2 * k] - (mlm[2 * k] - pcx) * inv_vw)
        lacc = lacc + _sl1(landmT_ref[0, 2 * k + 1] - (mlm[2 * k + 1] - pcy) * inv_vh)
    llm_img = jnp.sum(jnp.where(pos, lacc, 0.0))

    c0 = confT_ref[0, 0]
    c1 = confT_ref[0, 1]
    mx = jnp.maximum(c0, c1)
    lse = mx + jnp.log(jnp.exp(c0 - mx) + jnp.exp(c1 - mx))
    gath = jnp.where(pos, c1, c0)
    v = lse - gath
    lc_pos = jnp.sum(jnp.where(pos, v, 0.0))

    npos_i = jnp.sum(pos.astype(jnp.int32))
    k_sel = jnp.minimum(NEGPOS * npos_i, P - npos_i)

    vb_ref[b] = jnp.where(pos, -1, lax.bitcast_convert_type(v, jnp.int32))
    ks_ref[b] = k_sel

    out_ref[0] += ll_img
    out_ref[1] += lc_pos
    out_ref[2] += llm_img
    out_ref[3] += npos

    @pl.when(b == B - 1)
    def _final():
        ks = [ks_ref[i] for i in range(B)]
        k1 = [jnp.maximum(ks[i], 1) for i in range(B)]

        def bs_body(_, carry):
            lo, hi = carry
            nlo = []
            nhi = []
            for i in range(B):
                mid = lo[i] + (hi[i] - lo[i]) // 2
                cnt = jnp.sum((vb_ref[i] > mid).astype(jnp.int32))
                gek = cnt >= k1[i]
                nlo.append(jnp.where(gek, mid, lo[i]))
                nhi.append(jnp.where(gek, hi[i], mid))
            return tuple(nlo), tuple(nhi)

        lo0 = tuple(jnp.int32(-1) for _ in range(B))
        hi0 = tuple(jnp.int32(INF_BITS) for _ in range(B))
        _, hi = lax.fori_loop(0, 31, bs_body, (lo0, hi0))

        lc_neg = jnp.float32(0.0)
        for i in range(B):
            vb = vb_ref[i]
            t = lax.bitcast_convert_type(hi[i], jnp.float32)
            gt_mask = vb > hi[i]
            cnt_gt = jnp.sum(gt_mask.astype(jnp.int32))
            vi = lax.bitcast_convert_type(vb, jnp.float32)
            sum_gt = jnp.sum(jnp.where(gt_mask, vi, 0.0))
            topsum = sum_gt + (k1[i] - cnt_gt).astype(jnp.float32) * t
            lc_neg = lc_neg + jnp.where(ks[i] > 0, topsum, 0.0)

        n = jnp.maximum(out_ref[3], 1.0)
        out_ref[0] = out_ref[0] / n
        out_ref[1] = (out_ref[1] + lc_neg) / n
        out_ref[2] = out_ref[2] / n


@jax.jit
def _run(loc_data, conf_data, landm_data, priors, gt_bboxes, gt_landmarks,
         gt_num, img_shape):
    locT = loc_data.transpose(0, 2, 1).reshape(B, 4, R, L)
    confT = conf_data.transpose(0, 2, 1).reshape(B, 2, R, L)
    landmT = landm_data.transpose(0, 2, 1).reshape(B, 10, R, L)
    priT = priors.T.reshape(4, R, L)

    out = pl.pallas_call(
        _body,
        grid=(B,),
        in_specs=[
            pl.BlockSpec((1, G, 4), lambda b: (b, 0, 0), memory_space=pltpu.SMEM),
            pl.BlockSpec((1, G, 10), lambda b: (b, 0, 0), memory_space=pltpu.SMEM),
            pl.BlockSpec((1, 1, 1), lambda b: (b, 0, 0), memory_space=pltpu.SMEM),
            pl.BlockSpec((1, 1, 2), lambda b: (b, 0, 0), memory_space=pltpu.SMEM),
            pl.BlockSpec((1, 4, R, L), lambda b: (b, 0, 0, 0)),
            pl.BlockSpec((1, 2, R, L), lambda b: (b, 0, 0, 0)),
            pl.BlockSpec((1, 10, R, L), lambda b: (b, 0, 0, 0)),
            pl.BlockSpec((4, R, L), lambda b: (0, 0, 0)),
        ],
        out_specs=pl.BlockSpec(memory_space=pltpu.SMEM),
        out_shape=jax.ShapeDtypeStruct((4,), jnp.float32),
        scratch_shapes=[pltpu.VMEM((B, R, L), jnp.int32),
                        pltpu.SMEM((B,), jnp.int32),
                        pltpu.VMEM((R, L), jnp.float32),
                        pltpu.VMEM((R, L), jnp.int32),
                        pltpu.VMEM((R, L), jnp.int32),
                        pltpu.VMEM((R, L), jnp.int32)],
    )(gt_bboxes, gt_landmarks, gt_num.reshape(B, 1, 1),
      img_shape.reshape(B, 1, 2), locT, confT, landmT, priT)
    return out[0], out[1], out[2]


def kernel(loc_data, conf_data, landm_data, priors, gt_bboxes, gt_labels,
           gt_landmarks, gt_num, img_shape):
    del gt_labels
    return _run(loc_data, conf_data, landm_data, priors, gt_bboxes.astype(jnp.float32),
                gt_landmarks.astype(jnp.float32), gt_num, img_shape)


# speedup vs baseline: 3.8643x; 1.0800x over previous
import functools

import jax
import jax.numpy as jnp
from jax import lax
from jax.experimental import pallas as pl
from jax.experimental.pallas import tpu as pltpu

B = 16
P = 32768
G = 32
R = 256
L = 128
CH = 32
NC = R // CH
THR = 0.35
NEGPOS = 7
VAR0 = 0.1
VAR1 = 0.2
INF_BITS = 0x7F800000


def _sl1(d):
    a = jnp.abs(d)
    return jnp.where(a < 1.0, 0.5 * a * a, a - 0.5)


def _body(gtb_ref, gtl_ref, gtn_ref, img_ref, locT_ref, confT_ref, landmT_ref,
          priT_ref, out_ref, vb_ref, ks_ref, best_ref, bti_ref, forced_ref,
          jl_ref, m_ref):
    b = pl.program_id(0)

    @pl.when(b == 0)
    def _init():
        out_ref[0] = 0.0
        out_ref[1] = 0.0
        out_ref[2] = 0.0
        out_ref[3] = 0.0

    w = img_ref[0, 0, 1].astype(jnp.float32)
    h = img_ref[0, 0, 0].astype(jnp.float32)
    g = gtn_ref[0, 0, 0]

    pcx = priT_ref[0]
    pcy = priT_ref[1]
    pw = priT_ref[2]
    ph = priT_ref[3]
    px1 = pcx - pw * 0.5
    py1 = pcy - ph * 0.5
    px2 = pcx + pw * 0.5
    py2 = pcy + ph * 0.5
    parea = pw * ph

    lin = (lax.broadcasted_iota(jnp.int32, (R, L), 0) * L
           + lax.broadcasted_iota(jnp.int32, (R, L), 1))

    tx1 = [gtb_ref[0, j, 0] / w for j in range(G)]
    ty1 = [gtb_ref[0, j, 1] / h for j in range(G)]
    tx2 = [gtb_ref[0, j, 2] / w for j in range(G)]
    ty2 = [gtb_ref[0, j, 3] / h for j in range(G)]

    best_ref[...] = jnp.full((R, L), -2.0, jnp.float32)
    bti_ref[...] = jnp.zeros((R, L), jnp.int32)
    forced_ref[...] = jnp.zeros((R, L), jnp.int32)
    jl_ref[...] = jnp.full((R, L), -1, jnp.int32)
    for j in range(G):
        @pl.when(j < g)
        def _match(j=j):
            iw = jnp.maximum(jnp.minimum(tx2[j], px2) - jnp.maximum(tx1[j], px1), 0.0)
            ih = jnp.maximum(jnp.minimum(ty2[j], py2) - jnp.maximum(ty1[j], py1), 0.0)
            inter = iw * ih
            ta = (tx2[j] - tx1[j]) * (ty2[j] - ty1[j])
            iou = inter / (ta + parea - inter)
            bpo_j = jnp.max(iou)
            bpi_j = jnp.min(jnp.where(iou == bpo_j, lin, P))
            upd = iou > best_ref[...]
            best_ref[...] = jnp.where(upd, iou, best_ref[...])
            bti_ref[...] = jnp.where(upd, j, bti_ref[...])
            t1 = jnp.where(bpo_j >= 0.2, bpi_j, -1)
            forced_ref[...] = jnp.where(lin == t1, 1, forced_ref[...])
            jl_ref[...] = jnp.where(lin == bpi_j, j, jl_ref[...])

    best = best_ref[...]
    jl = jl_ref[...]
    bti = jnp.where(jl >= 0, jl, bti_ref[...])
    forced = forced_ref[...] != 0

    pos = (best >= THR) | forced
    npos = jnp.sum(pos.astype(jnp.float32))

    m_ref[...] = jnp.zeros((14, R, L), jnp.float32)
    for j in range(G):
        @pl.when(j < g)
        def _gather(j=j):
            sel = bti == j
            m_ref[0] = jnp.where(sel, (tx1[j] + tx2[j]) * 0.5, m_ref[0])
            m_ref[1] = jnp.where(sel, (ty1[j] + ty2[j]) * 0.5, m_ref[1])
            m_ref[2] = jnp.where(sel, tx2[j] - tx1[j], m_ref[2])
            m_ref[3] = jnp.where(sel, ty2[j] - ty1[j], m_ref[3])
            for k in range(5):
                m_ref[4 + 2 * k] = jnp.where(sel, gtl_ref[0, j, 2 * k] / w,
                                             m_ref[4 + 2 * k])
                m_ref[5 + 2 * k] = jnp.where(sel, gtl_ref[0, j, 2 * k + 1] / h,
                                             m_ref[5 + 2 * k])
    mcx = m_ref[0]
    mcy = m_ref[1]
    mw = m_ref[2]
    mh = m_ref[3]
    mlm = [m_ref[4 + k] for k in range(10)]

    inv_vw = 1.0 / (VAR0 * pw)
    inv_vh = 1.0 / (VAR0 * ph)
    acc = _sl1(locT_ref[0, 0] - (mcx - pcx) * inv_vw)
    acc = acc + _sl1(locT_ref[0, 1] - (mcy - pcy) * inv_vh)
    acc = acc + _sl1(locT_ref[0, 2] - jnp.log(mw / pw) * (1.0 / VAR1))
    acc = acc + _sl1(locT_ref[0, 3] - jnp.log(mh / ph) * (1.0 / VAR1))
    ll_img = jnp.sum(jnp.where(pos, acc, 0.0))

    lacc = jnp.zeros((R, L), jnp.float32)
    for k in range(5):
        lacc = lacc + _sl1(landmT_ref[0, 2 * k] - (mlm[2 * k] - pcx) * inv_vw)
        lacc = lacc + _sl1(landmT_ref[0, 2 * k + 1] - (mlm[2 * k + 1] - pcy) * inv_vh)
    llm_img = jnp.sum(jnp.where(pos, lacc, 0.0))

    c0 = confT_ref[0, 0]
    c1 = confT_ref[0, 1]
    mx = jnp.maximum(c0, c1)
    lse = mx + jnp.log(jnp.exp(c0 - mx) + jnp.exp(c1 - mx))
    gath = jnp.where(pos, c1, c0)
    v = lse - gath
    lc_pos = jnp.sum(jnp.where(pos, v, 0.0))

    npos_i = jnp.sum(pos.astype(jnp.int32))
    k_sel = jnp.minimum(NEGPOS * npos_i, P - npos_i)

    vb_ref[b] = jnp.where(pos, -1, lax.bitcast_convert_type(v, jnp.int32))
    ks_ref[b] = k_sel

    out_ref[0] += ll_img
    out_ref[1] += lc_pos
    out_ref[2] += llm_img
    out_ref[3] += npos

    @pl.when(b == B - 1)
    def _final():
        ks = [ks_ref[i] for i in range(B)]
        k1 = [jnp.maximum(ks[i], 1) for i in range(B)]

        def bs_body(_, carry):
            lo, hi = carry
            nlo = []
            nhi = []
            for i in range(B):
                mid = lo[i] + (hi[i] - lo[i]) // 2
                cnt = jnp.sum((vb_ref[i] > mid).astype(jnp.int32))
                gek = cnt >= k1[i]
                nlo.append(jnp.where(gek, mid, lo[i]))
                nhi.append(jnp.where(gek, hi[i], mid))
            return tuple(nlo), tuple(nhi)

        lo0 = tuple(jnp.int32(-1) for _ in range(B))
        hi0 = tuple(jnp.int32(INF_BITS) for _ in range(B))
        _, hi = lax.fori_loop(0, 31, bs_body, (lo0, hi0))

        lc_neg = jnp.float32(0.0)
        for i in range(B):
            vb = vb_ref[i]
            t = lax.bitcast_convert_type(hi[i], jnp.float32)
            gt_mask = vb > hi[i]
            cnt_gt = jnp.sum(gt_mask.astype(jnp.int32))
            vi = lax.bitcast_convert_type(vb, jnp.float32)
            sum_gt = jnp.sum(jnp.where(gt_mask, vi, 0.0))
            topsum = sum_gt + (k1[i] - cnt_gt).astype(jnp.float32) * t
            lc_neg = lc_neg + jnp.where(ks[i] > 0, topsum, 0.0)

        n = jnp.maximum(out_ref[3], 1.0)
        out_ref[0] = out_ref[0] / n
        out_ref[1] = (out_ref[1] + lc_neg) / n
        out_ref[2] = out_ref[2] / n


@jax.jit
def _run(loc_data, conf_data, landm_data, priors, gt_bboxes, gt_landmarks,
         gt_num, img_shape):
    locT = loc_data.transpose(0, 2, 1).reshape(B, 4, R, L)
    confT = conf_data.transpose(0, 2, 1).reshape(B, 2, R, L)
    landmT = landm_data.transpose(0, 2, 1).reshape(B, 10, R, L)
    priT = priors.T.reshape(4, R, L)

    out = pl.pallas_call(
        _body,
        grid=(B,),
        in_specs=[
            pl.BlockSpec((1, G, 4), lambda b: (b, 0, 0), memory_space=pltpu.SMEM),
            pl.BlockSpec((1, G, 10), lambda b: (b, 0, 0), memory_space=pltpu.SMEM),
            pl.BlockSpec((1, 1, 1), lambda b: (b, 0, 0), memory_space=pltpu.SMEM),
            pl.BlockSpec((1, 1, 2), lambda b: (b, 0, 0), memory_space=pltpu.SMEM),
            pl.BlockSpec((1, 4, R, L), lambda b: (b, 0, 0, 0)),
            pl.BlockSpec((1, 2, R, L), lambda b: (b, 0, 0, 0)),
            pl.BlockSpec((1, 10, R, L), lambda b: (b, 0, 0, 0)),
            pl.BlockSpec((4, R, L), lambda b: (0, 0, 0)),
        ],
        out_specs=pl.BlockSpec(memory_space=pltpu.SMEM),
        out_shape=jax.ShapeDtypeStruct((4,), jnp.float32),
        scratch_shapes=[pltpu.VMEM((B, R, L), jnp.int32),
                        pltpu.SMEM((B,), jnp.int32),
                        pltpu.VMEM((R, L), jnp.float32),
                        pltpu.VMEM((R, L), jnp.int32),
                        pltpu.VMEM((R, L), jnp.int32),
                        pltpu.VMEM((R, L), jnp.int32),
                        pltpu.VMEM((14, R, L), jnp.float32)],
    )(gt_bboxes, gt_landmarks, gt_num.reshape(B, 1, 1),
      img_shape.reshape(B, 1, 2), locT, confT, landmT, priT)
    return out[0], out[1], out[2]


def kernel(loc_data, conf_data, landm_data, priors, gt_bboxes, gt_labels,
           gt_landmarks, gt_num, img_shape):
    del gt_labels
    return _run(loc_data, conf_data, landm_data, priors, gt_bboxes.astype(jnp.float32),
                gt_landmarks.astype(jnp.float32), gt_num, img_shape)


# speedup vs baseline: 3.8707x; 1.0017x over previous
import functools

import jax
import jax.numpy as jnp
from jax import lax
from jax.experimental import pallas as pl
from jax.experimental.pallas import tpu as pltpu

B = 16
P = 32768
G = 32
R = 256
L = 128
THR = 0.35
NEGPOS = 7
VAR0 = 0.1
VAR1 = 0.2
INF_BITS = 0x7F800000


def _sl1(d):
    a = jnp.abs(d)
    return jnp.where(a < 1.0, 0.5 * a * a, a - 0.5)


def _body(gtb_ref, gtl_ref, gtn_ref, img_ref, locT_ref, confT_ref, landmT_ref,
          priT_ref, out_ref, vb_ref, ks_ref, best_ref, bti_ref, forced_ref,
          jl_ref, m_ref):
    b = pl.program_id(0)

    @pl.when(b == 0)
    def _init():
        out_ref[0] = 0.0
        out_ref[1] = 0.0
        out_ref[2] = 0.0
        out_ref[3] = 0.0

    w = img_ref[0, 0, 1].astype(jnp.float32)
    h = img_ref[0, 0, 0].astype(jnp.float32)
    g = gtn_ref[0, 0, 0]

    pcx = priT_ref[0]
    pcy = priT_ref[1]
    pw = priT_ref[2]
    ph = priT_ref[3]
    px1 = pcx - pw * 0.5
    py1 = pcy - ph * 0.5
    px2 = pcx + pw * 0.5
    py2 = pcy + ph * 0.5
    parea = pw * ph

    lin = (lax.broadcasted_iota(jnp.int32, (R, L), 0) * L
           + lax.broadcasted_iota(jnp.int32, (R, L), 1))

    tx1 = [gtb_ref[0, j, 0] / w for j in range(G)]
    ty1 = [gtb_ref[0, j, 1] / h for j in range(G)]
    tx2 = [gtb_ref[0, j, 2] / w for j in range(G)]
    ty2 = [gtb_ref[0, j, 3] / h for j in range(G)]

    best_ref[...] = jnp.full((R, L), -2.0, jnp.float32)
    bti_ref[...] = jnp.zeros((R, L), jnp.int32)
    forced_ref[...] = jnp.zeros((R, L), jnp.int32)
    jl_ref[...] = jnp.full((R, L), -1, jnp.int32)
    for j in range(G):
        @pl.when(j < g)
        def _match(j=j):
            iw = jnp.maximum(jnp.minimum(tx2[j], px2) - jnp.maximum(tx1[j], px1), 0.0)
            ih = jnp.maximum(jnp.minimum(ty2[j], py2) - jnp.maximum(ty1[j], py1), 0.0)
            inter = iw * ih
            ta = (tx2[j] - tx1[j]) * (ty2[j] - ty1[j])
            iou = inter / (ta + parea - inter)
            bpo_j = jnp.max(iou)
            bpi_j = jnp.min(jnp.where(iou == bpo_j, lin, P))
            upd = iou > best_ref[...]
            best_ref[...] = jnp.where(upd, iou, best_ref[...])
            bti_ref[...] = jnp.where(upd, j, bti_ref[...])
            t1 = jnp.where(bpo_j >= 0.2, bpi_j, -1)
            forced_ref[...] = jnp.where(lin == t1, 1, forced_ref[...])
            jl_ref[...] = jnp.where(lin == bpi_j, j, jl_ref[...])

    best = best_ref[...]
    jl = jl_ref[...]
    bti = jnp.where(jl >= 0, jl, bti_ref[...])
    forced = forced_ref[...] != 0

    pos = (best >= THR) | forced
    npos = jnp.sum(pos.astype(jnp.float32))

    m_ref[...] = jnp.zeros((14, R, L), jnp.float32)
    for j in range(G):
        @pl.when(j < g)
        def _gather(j=j):
            sel = bti == j
            m_ref[0] = jnp.where(sel, (tx1[j] + tx2[j]) * 0.5, m_ref[0])
            m_ref[1] = jnp.where(sel, (ty1[j] + ty2[j]) * 0.5, m_ref[1])
            m_ref[2] = jnp.where(sel, tx2[j] - tx1[j], m_ref[2])
            m_ref[3] = jnp.where(sel, ty2[j] - ty1[j], m_ref[3])
            for k in range(5):
                m_ref[4 + 2 * k] = jnp.where(sel, gtl_ref[0, j, 2 * k] / w,
                                             m_ref[4 + 2 * k])
                m_ref[5 + 2 * k] = jnp.where(sel, gtl_ref[0, j, 2 * k + 1] / h,
                                             m_ref[5 + 2 * k])
    mcx = m_ref[0]
    mcy = m_ref[1]
    mw = m_ref[2]
    mh = m_ref[3]
    mlm = [m_ref[4 + k] for k in range(10)]

    inv_vw = 1.0 / (VAR0 * pw)
    inv_vh = 1.0 / (VAR0 * ph)
    acc = _sl1(locT_ref[0, 0] - (mcx - pcx) * inv_vw)
    acc = acc + _sl1(locT_ref[0, 1] - (mcy - pcy) * inv_vh)
    acc = acc + _sl1(locT_ref[0, 2] - jnp.log(mw / pw) * (1.0 / VAR1))
    acc = acc + _sl1(locT_ref[0, 3] - jnp.log(mh / ph) * (1.0 / VAR1))
    ll_img = jnp.sum(jnp.where(pos, acc, 0.0))

    lacc = jnp.zeros((R, L), jnp.float32)
    for k in range(5):
        lacc = lacc + _sl1(landmT_ref[0, 2 * k] - (mlm[2 * k] - pcx) * inv_vw)
        lacc = lacc + _sl1(landmT_ref[0, 2 * k + 1] - (mlm[2 * k + 1] - pcy) * inv_vh)
    llm_img = jnp.sum(jnp.where(pos, lacc, 0.0))

    c0 = confT_ref[0, 0]
    c1 = confT_ref[0, 1]
    mx = jnp.maximum(c0, c1)
    lse = mx + jnp.log(jnp.exp(c0 - mx) + jnp.exp(c1 - mx))
    gath = jnp.where(pos, c1, c0)
    v = lse - gath
    lc_pos = jnp.sum(jnp.where(pos, v, 0.0))

    npos_i = jnp.sum(pos.astype(jnp.int32))
    k_sel = jnp.minimum(NEGPOS * npos_i, P - npos_i)

    vb_ref[b] = jnp.where(pos, -1, lax.bitcast_convert_type(v, jnp.int32))
    ks_ref[b] = k_sel

    out_ref[0] += ll_img
    out_ref[1] += lc_pos
    out_ref[2] += llm_img
    out_ref[3] += npos

    @pl.when(b == B - 1)
    def _final():
        ks = [ks_ref[i] for i in range(B)]
        k1 = [jnp.maximum(ks[i], 1) for i in range(B)]

        def bs_body(_, carry):
            lo, hi = carry
            nlo = []
            nhi = []
            for i in range(B):
                mid = lo[i] + (hi[i] - lo[i]) // 2
                cnt = jnp.sum((vb_ref[i] > mid).astype(jnp.int32))
                gek = cnt >= k1[i]
                nlo.append(jnp.where(gek, mid, lo[i]))
                nhi.append(jnp.where(gek, hi[i], mid))
            return tuple(nlo), tuple(nhi)

        lo0 = tuple(jnp.int32(-1) for _ in range(B))
        hi0 = tuple(jnp.int32(INF_BITS) for _ in range(B))
        _, hi = lax.fori_loop(0, 31, bs_body, (lo0, hi0))

        lc_neg = jnp.float32(0.0)
        for i in range(B):
            vb = vb_ref[i]
            t = lax.bitcast_convert_type(hi[i], jnp.float32)
            gt_mask = vb > hi[i]
            cnt_gt = jnp.sum(gt_mask.astype(jnp.int32))
            vi = lax.bitcast_convert_type(vb, jnp.float32)
            sum_gt = jnp.sum(jnp.where(gt_mask, vi, 0.0))
            topsum = sum_gt + (k1[i] - cnt_gt).astype(jnp.float32) * t
            lc_neg = lc_neg + jnp.where(ks[i] > 0, topsum, 0.0)

        n = jnp.maximum(out_ref[3], 1.0)
        out_ref[0] = out_ref[0] / n
        out_ref[1] = (out_ref[1] + lc_neg) / n
        out_ref[2] = out_ref[2] / n


@jax.jit
def _run(loc_data, conf_data, landm_data, priors, gt_bboxes, gt_landmarks,
         gt_num, img_shape):
    locT = loc_data.transpose(0, 2, 1).reshape(B, 4, R, L)
    confT = conf_data.transpose(0, 2, 1).reshape(B, 2, R, L)
    landmT = landm_data.transpose(0, 2, 1).reshape(B, 10, R, L)
    priT = priors.T.reshape(4, R, L)

    out = pl.pallas_call(
        _body,
        grid=(B,),
        in_specs=[
            pl.BlockSpec((1, G, 4), lambda b: (b, 0, 0), memory_space=pltpu.SMEM),
            pl.BlockSpec((1, G, 10), lambda b: (b, 0, 0), memory_space=pltpu.SMEM),
            pl.BlockSpec((1, 1, 1), lambda b: (b, 0, 0), memory_space=pltpu.SMEM),
            pl.BlockSpec((1, 1, 2), lambda b: (b, 0, 0), memory_space=pltpu.SMEM),
            pl.BlockSpec((1, 4, R, L), lambda b: (b, 0, 0, 0)),
            pl.BlockSpec((1, 2, R, L), lambda b: (b, 0, 0, 0)),
            pl.BlockSpec((1, 10, R, L), lambda b: (b, 0, 0, 0)),
            pl.BlockSpec((4, R, L), lambda b: (0, 0, 0)),
        ],
        out_specs=pl.BlockSpec(memory_space=pltpu.SMEM),
        out_shape=jax.ShapeDtypeStruct((4,), jnp.float32),
        scratch_shapes=[pltpu.VMEM((B, R, L), jnp.int32),
                        pltpu.SMEM((B,), jnp.int32),
                        pltpu.VMEM((R, L), jnp.float32),
                        pltpu.VMEM((R, L), jnp.int32),
                        pltpu.VMEM((R, L), jnp.int32),
                        pltpu.VMEM((R, L), jnp.int32),
                        pltpu.VMEM((14, R, L), jnp.float32)],
    )(gt_bboxes, gt_landmarks, gt_num.reshape(B, 1, 1),
      img_shape.reshape(B, 1, 2), locT, confT, landmT, priT)
    return out[0], out[1], out[2]


def kernel(loc_data, conf_data, landm_data, priors, gt_bboxes, gt_labels,
           gt_landmarks, gt_num, img_shape):
    del gt_labels
    return _run(loc_data, conf_data, landm_data, priors, gt_bboxes.astype(jnp.float32),
                gt_landmarks.astype(jnp.float32), gt_num, img_shape)
